# baseline (speedup 1.0000x reference)
import functools

import jax
import jax.numpy as jnp
from jax.experimental import pallas as pl
from jax.experimental.pallas import tpu as pltpu

EPS = 1e-6
CT = 64
G = 16
TB = 512
TBC = 512


def _silu(z):
    return z * jax.nn.sigmoid(z)


def _proj_kernel(H, HD, x_ref, wq_ref, wk_ref, wv_ref, wf1_ref, wf2_ref,
                 wb_ref, wog1_ref, wog2_ref,
                 q_ref, k_ref, kmb_ref, v_ref, lf_ref, g_ref):
    bf16 = jnp.bfloat16
    f32 = jnp.float32
    x = x_ref[0]
    q = _silu(jnp.dot(x, wq_ref[...], preferred_element_type=f32))
    kx = _silu(jnp.dot(x, wk_ref[...], preferred_element_type=f32))
    v = _silu(jnp.dot(x, wv_ref[...], preferred_element_type=f32))
    f = jnp.dot(jnp.dot(x, wf1_ref[...], preferred_element_type=f32
                        ).astype(bf16),
                wf2_ref[...], preferred_element_type=f32)
    lf = jax.nn.log_sigmoid(f)
    beta = jax.nn.sigmoid(jnp.dot(x, wb_ref[...],
                                  preferred_element_type=f32)) * 2.0
    g = jax.nn.sigmoid(jnp.dot(jnp.dot(x, wog1_ref[...],
                                       preferred_element_type=f32
                                       ).astype(bf16),
                               wog2_ref[...], preferred_element_type=f32))
    for h in range(H):
        sl = slice(h * HD, (h + 1) * HD)
        kh = kx[:, sl]
        khn = kh / jnp.sqrt(jnp.sum(kh * kh, axis=-1, keepdims=True) + EPS)
        q_ref[h] = q[:, sl]
        k_ref[h] = khn
        kmb_ref[h] = -beta[:, h:h + 1] * khn
        v_ref[h] = v[:, sl]
        lf_ref[h] = lf[:, sl]
        g_ref[h] = g[:, sl]


def _scan_kernel(ct, hd, g, q_ref, k_ref, kmb_ref, v_ref, lf_ref, o_ref, s_ref):
    c = pl.program_id(1)

    @pl.when(c == 0)
    def _():
        s_ref[...] = jnp.zeros_like(s_ref)

    f32 = jnp.float32
    cc = 2 * ct

    rt = jax.lax.broadcasted_iota(jnp.int32, (ct, ct), 0)
    ctco = jax.lax.broadcasted_iota(jnp.int32, (ct, ct), 1)
    tri = jnp.where(ctco <= rt, 1.0, 0.0).astype(f32)
    r2 = jax.lax.broadcasted_iota(jnp.int32, (cc, cc), 0)
    c2 = jax.lax.broadcasted_iota(jnp.int32, (cc, cc), 1)
    tr = jnp.where(r2 < ct, 2 * r2, 2 * r2 - (cc - 1))
    tc = jnp.where(c2 < ct, 2 * c2, 2 * c2 - (cc - 1))
    mask_la = tc < tr
    eye = jnp.where(r2 == c2, 1.0, 0.0)
    rK = jax.lax.broadcasted_iota(jnp.int32, (cc, ct), 0)
    cK = jax.lax.broadcasted_iota(jnp.int32, (cc, ct), 1)
    trK = jnp.where(rK < ct, 2 * rK, 2 * rK - (cc - 1))
    mask_lk = 2 * cK + 1 < trK
    rO = jax.lax.broadcasted_iota(jnp.int32, (ct, cc), 0)
    cO = jax.lax.broadcasted_iota(jnp.int32, (ct, cc), 1)
    tcO = jnp.where(cO < ct, 2 * cO, 2 * cO - (cc - 1))
    mask_qa = tcO <= 2 * rO + 1
    mask_qk = ctco <= rt

    def dot_tt(a, b):
        return jax.lax.dot_general(a, b, (((1,), (1,)), ((), ())),
                                   preferred_element_type=f32)

    def dot_ff(a, b):
        return jax.lax.dot_general(a, b, (((0,), (0,)), ((), ())),
                                   preferred_element_type=f32)

    bf16 = jnp.bfloat16

    def mm(a, b):
        return jnp.dot(a, b, preferred_element_type=f32)

    gr = range(g)
    Ss = [s_ref[gi] for gi in gr]

    At, Bt, Kt, Qt, vb, Sb, gam = [], [], [], [], [], [], []
    for gi in gr:
        lf = lf_ref[gi]
        k = k_ref[gi]
        kmb = kmb_ref[gi]
        F = jnp.dot(tri, lf, preferred_element_type=f32)
        Fex = F - lf
        epF = jnp.exp(F)
        enF = jnp.exp(-F)
        epX = jnp.exp(Fex)
        enX = jnp.exp(-Fex)
        At.append(jnp.concatenate([kmb * enX, kmb * enF], axis=0).astype(bf16))
        Bt.append(jnp.concatenate([k * epX, k * epF], axis=0).astype(bf16))
        Kt.append((k * enF).astype(bf16))
        Qt.append((q_ref[gi] * epF).astype(bf16))
        vb.append(v_ref[gi].astype(bf16))
        Sb.append(Ss[gi].astype(bf16))
        gam.append(epF[ct - 1:ct, :])

    Pm = [dot_tt(jnp.concatenate([Bt[gi], Qt[gi]], axis=0),
                 jnp.concatenate([At[gi], Kt[gi], Sb[gi]], axis=0))
          for gi in gr]
    LA = [jnp.where(mask_la, Pm[gi][:cc, :cc], 0.0).astype(bf16) for gi in gr]
    LK = [jnp.where(mask_lk, Pm[gi][:cc, cc:cc + ct], 0.0).astype(bf16) for gi in gr]
    QA = [jnp.where(mask_qa, Pm[gi][cc:, :cc], 0.0).astype(bf16) for gi in gr]
    QK = [jnp.where(mask_qk, Pm[gi][cc:, cc:cc + ct], 0.0).astype(bf16) for gi in gr]
    KV = [mm(jnp.concatenate([LK[gi], QK[gi]], axis=0), vb[gi]) for gi in gr]
    R = [(Pm[gi][:cc, cc + ct:] + KV[gi][:cc]).astype(bf16) for gi in gr]

    L2 = [mm(LA[gi], LA[gi]).astype(bf16) for gi in gr]
    L4 = [mm(L2[gi], L2[gi]).astype(bf16) for gi in gr]
    L8 = [mm(L4[gi], L4[gi]).astype(bf16) for gi in gr]
    L16 = [mm(L8[gi], L8[gi]).astype(bf16) for gi in gr]
    L32 = [mm(L16[gi], L16[gi]).astype(bf16) for gi in gr]
    L64 = [mm(L32[gi], L32[gi]).astype(bf16) for gi in gr]
    G1 = [(eye + LA[gi].astype(f32) + L2[gi].astype(f32)
           + mm(LA[gi], L2[gi])).astype(bf16) for gi in gr]
    G2 = [(eye + L4[gi].astype(f32) + L8[gi].astype(f32)
           + mm(L4[gi], L8[gi])).astype(bf16) for gi in gr]
    G3 = [(eye + L16[gi].astype(f32) + L32[gi].astype(f32)
           + mm(L16[gi], L32[gi])).astype(bf16) for gi in gr]
    H1 = [mm(G1[gi], G2[gi]).astype(bf16) for gi in gr]
    H2 = [(G3[gi].astype(f32) + L64[gi].astype(f32)
           + mm(G3[gi], L64[gi])).astype(bf16) for gi in gr]
    T = [mm(H1[gi], H2[gi]).astype(bf16) for gi in gr]
    U = [mm(T[gi], R[gi]).astype(bf16) for gi in gr]

    for gi in gr:
        o_ref[gi] = Pm[gi][cc:, cc + ct:] + mm(QA[gi], U[gi]) + KV[gi][cc:]
    for gi in gr:
        upd = dot_ff(jnp.concatenate([U[gi], vb[gi]], axis=0),
                     jnp.concatenate([At[gi], Kt[gi]], axis=0))
        s_ref[gi] = (Ss[gi] + upd) * gam[gi]


def _out_kernel(H, HD, o_ref, g_ref, nw_ref, wo_ref, y_ref):
    cols = []
    for h in range(H):
        yh = o_ref[h] * g_ref[h]
        yh = yh / jnp.sqrt(jnp.mean(yh * yh, axis=-1, keepdims=True) + EPS)
        cols.append(yh * nw_ref[h:h + 1, :])
    y = jnp.concatenate(cols, axis=-1).astype(jnp.bfloat16)
    y_ref[0] = jnp.dot(y, wo_ref[...], preferred_element_type=jnp.float32)


def kernel(x, Wq, Wk, Wv, Wf1, Wf2, Wbeta, Wog1, Wog2, norm_w, Wo):
    B, N, D = x.shape
    H = Wbeta.shape[1]
    HD = D // H
    BH = B * H
    tb = min(TB, N)
    tbc = min(TBC, N)
    nt = N // tb
    f32 = jnp.float32
    wspec = pl.BlockSpec(memory_space=pltpu.VMEM)
    hspec = pl.BlockSpec((H, tb, HD), lambda b, t: (b, t, 0))
    sds = jax.ShapeDtypeStruct((BH, N, HD), f32)

    q, k, kmb, v, lf, g = pl.pallas_call(
        functools.partial(_proj_kernel, H, HD),
        grid=(B, nt),
        in_specs=[pl.BlockSpec((1, tb, D), lambda b, t: (b, t, 0))]
        + [wspec] * 8,
        out_specs=[hspec] * 6,
        out_shape=[sds] * 6,
        compiler_params=pltpu.CompilerParams(
            dimension_semantics=("parallel", "parallel"),
            vmem_limit_bytes=60 * 1024 * 1024,
        ),
        name="dense_rnn_proj",
    )(x.astype(jnp.bfloat16), Wq.astype(jnp.bfloat16), Wk.astype(jnp.bfloat16),
      Wv.astype(jnp.bfloat16), Wf1.astype(jnp.bfloat16),
      Wf2.astype(jnp.bfloat16), Wbeta.astype(jnp.bfloat16),
      Wog1.astype(jnp.bfloat16), Wog2.astype(jnp.bfloat16))

    nc = N // CT
    cspec = pl.BlockSpec((G, CT, HD), lambda bh, c: (bh, c, 0))
    o = pl.pallas_call(
        functools.partial(_scan_kernel, CT, HD, G),
        grid=(BH // G, nc),
        in_specs=[cspec] * 5,
        out_specs=cspec,
        out_shape=sds,
        scratch_shapes=[pltpu.VMEM((G, HD, HD), f32)],
        compiler_params=pltpu.CompilerParams(
            dimension_semantics=("parallel", "arbitrary"),
            vmem_limit_bytes=60 * 1024 * 1024,
        ),
        name="dense_rnn_scan",
    )(q, k, kmb, v, lf)

    ntc = N // tbc
    y = pl.pallas_call(
        functools.partial(_out_kernel, H, HD),
        grid=(B, ntc),
        in_specs=[pl.BlockSpec((H, tbc, HD), lambda b, t: (b, t, 0))] * 2
        + [wspec, wspec],
        out_specs=pl.BlockSpec((1, tbc, D), lambda b, t: (b, t, 0)),
        out_shape=jax.ShapeDtypeStruct((B, N, D), f32),
        compiler_params=pltpu.CompilerParams(
            dimension_semantics=("parallel", "parallel"),
            vmem_limit_bytes=60 * 1024 * 1024,
        ),
        name="dense_rnn_out",
    )(o, g, norm_w.reshape(H, HD), Wo.astype(jnp.bfloat16))
    return y


# speedup vs baseline: 1.0766x; 1.0766x over previous
import functools

import jax
import jax.numpy as jnp
from jax.experimental import pallas as pl
from jax.experimental.pallas import tpu as pltpu

EPS = 1e-6
CT = 64
G = 16
TB = 512
TBC = 512


def _silu(z):
    return z * jax.nn.sigmoid(z)


def _proj_kernel(H, HD, x_ref, wq_ref, wk_ref, wv_ref, wf1_ref, wf2_ref,
                 wb_ref, wog1_ref, wog2_ref,
                 q_ref, k_ref, kmb_ref, v_ref, lf_ref, g_ref):
    bf16 = jnp.bfloat16
    f32 = jnp.float32
    x = x_ref[0].astype(bf16)
    wq = wq_ref[...].astype(bf16)
    wk = wk_ref[...].astype(bf16)
    wv = wv_ref[...].astype(bf16)
    q = _silu(jnp.dot(x, wq, preferred_element_type=f32))
    kx = _silu(jnp.dot(x, wk, preferred_element_type=f32))
    v = _silu(jnp.dot(x, wv, preferred_element_type=f32))
    f = jnp.dot(jnp.dot(x, wf1_ref[...].astype(bf16),
                        preferred_element_type=f32).astype(bf16),
                wf2_ref[...].astype(bf16), preferred_element_type=f32)
    lf = jax.nn.log_sigmoid(f)
    beta = jax.nn.sigmoid(jnp.dot(x, wb_ref[...].astype(bf16),
                                  preferred_element_type=f32)) * 2.0
    g = jax.nn.sigmoid(jnp.dot(jnp.dot(x, wog1_ref[...].astype(bf16),
                                       preferred_element_type=f32).astype(bf16),
                               wog2_ref[...].astype(bf16),
                               preferred_element_type=f32))
    for h in range(H):
        sl = slice(h * HD, (h + 1) * HD)
        kh = kx[:, sl]
        khn = kh / jnp.sqrt(jnp.sum(kh * kh, axis=-1, keepdims=True) + EPS)
        q_ref[h] = q[:, sl]
        k_ref[h] = khn
        kmb_ref[h] = -beta[:, h:h + 1] * khn
        v_ref[h] = v[:, sl]
        lf_ref[h] = lf[:, sl]
        g_ref[h] = g[:, sl]


def _scan_kernel(ct, hd, g, q_ref, k_ref, kmb_ref, v_ref, lf_ref, o_ref, s_ref):
    c = pl.program_id(1)

    @pl.when(c == 0)
    def _():
        s_ref[...] = jnp.zeros_like(s_ref)

    f32 = jnp.float32
    cc = 2 * ct

    rt = jax.lax.broadcasted_iota(jnp.int32, (ct, ct), 0)
    ctco = jax.lax.broadcasted_iota(jnp.int32, (ct, ct), 1)
    tri = jnp.where(ctco <= rt, 1.0, 0.0).astype(f32)
    r2 = jax.lax.broadcasted_iota(jnp.int32, (cc, cc), 0)
    c2 = jax.lax.broadcasted_iota(jnp.int32, (cc, cc), 1)
    tr = jnp.where(r2 < ct, 2 * r2, 2 * r2 - (cc - 1))
    tc = jnp.where(c2 < ct, 2 * c2, 2 * c2 - (cc - 1))
    mask_la = tc < tr
    eye = jnp.where(r2 == c2, 1.0, 0.0)
    rK = jax.lax.broadcasted_iota(jnp.int32, (cc, ct), 0)
    cK = jax.lax.broadcasted_iota(jnp.int32, (cc, ct), 1)
    trK = jnp.where(rK < ct, 2 * rK, 2 * rK - (cc - 1))
    mask_lk = 2 * cK + 1 < trK
    rO = jax.lax.broadcasted_iota(jnp.int32, (ct, cc), 0)
    cO = jax.lax.broadcasted_iota(jnp.int32, (ct, cc), 1)
    tcO = jnp.where(cO < ct, 2 * cO, 2 * cO - (cc - 1))
    mask_qa = tcO <= 2 * rO + 1
    mask_qk = ctco <= rt

    def dot_tt(a, b):
        return jax.lax.dot_general(a, b, (((1,), (1,)), ((), ())),
                                   preferred_element_type=f32)

    def dot_ff(a, b):
        return jax.lax.dot_general(a, b, (((0,), (0,)), ((), ())),
                                   preferred_element_type=f32)

    bf16 = jnp.bfloat16

    def mm(a, b):
        return jnp.dot(a, b, preferred_element_type=f32)

    gr = range(g)
    Ss = [s_ref[gi] for gi in gr]

    At, Bt, Kt, Qt, vb, Sb, gam = [], [], [], [], [], [], []
    for gi in gr:
        lf = lf_ref[gi]
        k = k_ref[gi]
        kmb = kmb_ref[gi]
        F = jnp.dot(tri, lf, preferred_element_type=f32)
        Fex = F - lf
        epF = jnp.exp(F)
        enF = jnp.exp(-F)
        epX = jnp.exp(Fex)
        enX = jnp.exp(-Fex)
        At.append(jnp.concatenate([kmb * enX, kmb * enF], axis=0).astype(bf16))
        Bt.append(jnp.concatenate([k * epX, k * epF], axis=0).astype(bf16))
        Kt.append((k * enF).astype(bf16))
        Qt.append((q_ref[gi] * epF).astype(bf16))
        vb.append(v_ref[gi].astype(bf16))
        Sb.append(Ss[gi].astype(bf16))
        gam.append(epF[ct - 1:ct, :])

    Pm = [dot_tt(jnp.concatenate([Bt[gi], Qt[gi]], axis=0),
                 jnp.concatenate([At[gi], Kt[gi], Sb[gi]], axis=0))
          for gi in gr]
    LA = [jnp.where(mask_la, Pm[gi][:cc, :cc], 0.0).astype(bf16) for gi in gr]
    LK = [jnp.where(mask_lk, Pm[gi][:cc, cc:cc + ct], 0.0).astype(bf16) for gi in gr]
    QA = [jnp.where(mask_qa, Pm[gi][cc:, :cc], 0.0).astype(bf16) for gi in gr]
    QK = [jnp.where(mask_qk, Pm[gi][cc:, cc:cc + ct], 0.0).astype(bf16) for gi in gr]
    KV = [mm(jnp.concatenate([LK[gi], QK[gi]], axis=0), vb[gi]) for gi in gr]
    R = [(Pm[gi][:cc, cc + ct:] + KV[gi][:cc]).astype(bf16) for gi in gr]

    L2 = [mm(LA[gi], LA[gi]).astype(bf16) for gi in gr]
    L4 = [mm(L2[gi], L2[gi]).astype(bf16) for gi in gr]
    L8 = [mm(L4[gi], L4[gi]).astype(bf16) for gi in gr]
    L16 = [mm(L8[gi], L8[gi]).astype(bf16) for gi in gr]
    L32 = [mm(L16[gi], L16[gi]).astype(bf16) for gi in gr]
    L64 = [mm(L32[gi], L32[gi]).astype(bf16) for gi in gr]
    G1 = [(eye + LA[gi].astype(f32) + L2[gi].astype(f32)
           + mm(LA[gi], L2[gi])).astype(bf16) for gi in gr]
    G2 = [(eye + L4[gi].astype(f32) + L8[gi].astype(f32)
           + mm(L4[gi], L8[gi])).astype(bf16) for gi in gr]
    G3 = [(eye + L16[gi].astype(f32) + L32[gi].astype(f32)
           + mm(L16[gi], L32[gi])).astype(bf16) for gi in gr]
    H1 = [mm(G1[gi], G2[gi]).astype(bf16) for gi in gr]
    H2 = [(G3[gi].astype(f32) + L64[gi].astype(f32)
           + mm(G3[gi], L64[gi])).astype(bf16) for gi in gr]
    T = [mm(H1[gi], H2[gi]).astype(bf16) for gi in gr]
    U = [mm(T[gi], R[gi]).astype(bf16) for gi in gr]

    for gi in gr:
        o_ref[gi] = Pm[gi][cc:, cc + ct:] + mm(QA[gi], U[gi]) + KV[gi][cc:]
    for gi in gr:
        upd = dot_ff(jnp.concatenate([U[gi], vb[gi]], axis=0),
                     jnp.concatenate([At[gi], Kt[gi]], axis=0))
        s_ref[gi] = (Ss[gi] + upd) * gam[gi]


def _out_kernel(H, HD, o_ref, g_ref, nw_ref, wo_ref, y_ref):
    cols = []
    for h in range(H):
        yh = o_ref[h] * g_ref[h]
        yh = yh / jnp.sqrt(jnp.mean(yh * yh, axis=-1, keepdims=True) + EPS)
        cols.append(yh * nw_ref[h:h + 1, :])
    y = jnp.concatenate(cols, axis=-1).astype(jnp.bfloat16)
    y_ref[0] = jnp.dot(y, wo_ref[...].astype(jnp.bfloat16),
                       preferred_element_type=jnp.float32)


def kernel(x, Wq, Wk, Wv, Wf1, Wf2, Wbeta, Wog1, Wog2, norm_w, Wo):
    B, N, D = x.shape
    H = Wbeta.shape[1]
    HD = D // H
    BH = B * H
    tb = min(TB, N)
    tbc = min(TBC, N)
    nt = N // tb
    f32 = jnp.float32
    wspec = pl.BlockSpec(memory_space=pltpu.VMEM)
    hspec = pl.BlockSpec((H, tb, HD), lambda b, t: (b, t, 0))
    sds = jax.ShapeDtypeStruct((BH, N, HD), f32)

    q, k, kmb, v, lf, g = pl.pallas_call(
        functools.partial(_proj_kernel, H, HD),
        grid=(B, nt),
        in_specs=[pl.BlockSpec((1, tb, D), lambda b, t: (b, t, 0))]
        + [wspec] * 8,
        out_specs=[hspec] * 6,
        out_shape=[sds] * 6,
        compiler_params=pltpu.CompilerParams(
            dimension_semantics=("parallel", "parallel"),
            vmem_limit_bytes=60 * 1024 * 1024,
        ),
        name="dense_rnn_proj",
    )(x, Wq, Wk, Wv, Wf1, Wf2, Wbeta, Wog1, Wog2)

    nc = N // CT
    cspec = pl.BlockSpec((G, CT, HD), lambda bh, c: (bh, c, 0))
    o = pl.pallas_call(
        functools.partial(_scan_kernel, CT, HD, G),
        grid=(BH // G, nc),
        in_specs=[cspec] * 5,
        out_specs=cspec,
        out_shape=sds,
        scratch_shapes=[pltpu.VMEM((G, HD, HD), f32)],
        compiler_params=pltpu.CompilerParams(
            dimension_semantics=("parallel", "arbitrary"),
            vmem_limit_bytes=60 * 1024 * 1024,
        ),
        name="dense_rnn_scan",
    )(q, k, kmb, v, lf)

    ntc = N // tbc
    y = pl.pallas_call(
        functools.partial(_out_kernel, H, HD),
        grid=(B, ntc),
        in_specs=[pl.BlockSpec((H, tbc, HD), lambda b, t: (b, t, 0))] * 2
        + [wspec, wspec],
        out_specs=pl.BlockSpec((1, tbc, D), lambda b, t: (b, t, 0)),
        out_shape=jax.ShapeDtypeStruct((B, N, D), f32),
        compiler_params=pltpu.CompilerParams(
            dimension_semantics=("parallel", "parallel"),
            vmem_limit_bytes=60 * 1024 * 1024,
        ),
        name="dense_rnn_out",
    )(o, g, norm_w.reshape(H, HD), Wo)
    return y
